# baseline (speedup 1.0000x reference)
import functools

import jax
import jax.numpy as jnp
from jax import lax
from jax.experimental import pallas as pl
from jax.experimental.pallas import tpu as pltpu
from jax.experimental.pallas import tpu_sc as plsc

N = 10000
E = 320000
D = 128
DH = D // 2

NC = 2
NS = 16
CH = 256
CHZ = 128
NCHUNK = 40
EP = NC * NS * NCHUNK * CH
R = 10240
RT = R // NS

_MESH = plsc.VectorSubcoreMesh(core_axis_name="c", subcore_axis_name="s")


@functools.partial(
    pl.kernel,
    out_type=jax.ShapeDtypeStruct((NC, R, 16), jnp.float32),
    mesh=_MESH,
    scratch_types=[
        pltpu.VMEM((NCHUNK, CH), jnp.int32),
        pltpu.VMEM((CH, 16), jnp.float32),
        pltpu.VMEM((CHZ, 16), jnp.float32),
        pltpu.VMEM_SHARED((R, 16), jnp.float32),
    ],
    compiler_params=pltpu.CompilerParams(use_tc_tiling_on_sc=False),
)
def _deg_kernel(dst_hbm, out_hbm, dst_v, ones_v, zb_v, acc_sh):
    c = lax.axis_index("c")
    s = lax.axis_index("s")
    e0 = jnp.where(lax.iota(jnp.int32, 16) == 0,
                   jnp.float32(1.0), jnp.float32(0.0))
    z16 = jnp.zeros((16,), jnp.float32)

    def init_row(j, _):
        ones_v[j, :] = e0
        return 0
    lax.fori_loop(0, CH, init_row, 0)

    def init_zrow(j, _):
        zb_v[j, :] = z16
        return 0
    lax.fori_loop(0, CHZ, init_zrow, 0)

    base = s * RT
    def zcp(k, _):
        pltpu.sync_copy(zb_v, acc_sh.at[pl.ds(base + k * CHZ, CHZ)])
        return 0
    lax.fori_loop(0, RT // CHZ, zcp, 0)
    plsc.subcore_barrier()

    pltpu.sync_copy(dst_hbm.at[c, s], dst_v)

    def scat(j, _):
        pltpu.sync_copy(ones_v, acc_sh.at[dst_v.at[j]], add=True)
        return 0
    lax.fori_loop(0, NCHUNK, scat, 0)
    plsc.subcore_barrier()

    pltpu.sync_copy(acc_sh.at[pl.ds(base, RT)], out_hbm.at[c, pl.ds(base, RT)])


@functools.partial(
    pl.kernel,
    out_type=jax.ShapeDtypeStruct((NC, 2, R, DH), jnp.float32),
    mesh=_MESH,
    scratch_types=[
        pltpu.VMEM((NCHUNK, CH), jnp.int32),
        pltpu.VMEM((NCHUNK, CH), jnp.int32),
        pltpu.VMEM((CH, DH), jnp.float32),
        pltpu.VMEM((CH, DH), jnp.float32),
        pltpu.VMEM((CHZ, DH), jnp.float32),
        pltpu.VMEM_SHARED((R, DH), jnp.float32),
        pltpu.SemaphoreType.DMA,
        pltpu.SemaphoreType.DMA,
    ],
    compiler_params=pltpu.CompilerParams(use_tc_tiling_on_sc=False),
)
def _agg_kernel(ta_hbm, tb_hbm, src_hbm, dst_hbm, out_hbm,
                src_v, dst_v, rb0, rb1, zb_v, acc_sh, sem0, sem1):
    c = lax.axis_index("c")
    s = lax.axis_index("s")
    z16 = jnp.zeros((16,), jnp.float32)

    def zrow(j, _):
        for k in range(DH // 16):
            zb_v[j, pl.ds(k * 16, 16)] = z16
        return 0
    lax.fori_loop(0, CHZ, zrow, 0)

    base = s * RT
    pltpu.sync_copy(src_hbm.at[c, s], src_v)
    pltpu.sync_copy(dst_hbm.at[c, s], dst_v)

    for phase, table in ((0, ta_hbm), (1, tb_hbm)):
        def zcp(k, _):
            pltpu.sync_copy(zb_v, acc_sh.at[pl.ds(base + k * CHZ, CHZ)])
            return 0
        lax.fori_loop(0, RT // CHZ, zcp, 0)
        plsc.subcore_barrier()

        pltpu.async_copy(table.at[src_v.at[0]], rb0, sem0)
        pltpu.async_copy(table.at[src_v.at[1]], rb1, sem1)

        def step(t, _):
            g = t * 2
            for b, (rb, sem) in enumerate(((rb0, sem0), (rb1, sem1))):
                j = g + b
                pltpu.make_async_copy(table.at[src_v.at[j]], rb, sem).wait()
                pltpu.sync_copy(rb, acc_sh.at[dst_v.at[j]], add=True)

                @pl.when(j + 2 < NCHUNK)
                def _():
                    pltpu.async_copy(table.at[src_v.at[j + 2]], rb, sem)
            return 0
        lax.fori_loop(0, NCHUNK // 2, step, 0)
        plsc.subcore_barrier()

        pltpu.sync_copy(acc_sh.at[pl.ds(base, RT)],
                        out_hbm.at[c, phase, pl.ds(base, RT)])


_RB = 1000


def _ds_from_degp(degp_blk):
    deg = jnp.sum(degp_blk, axis=(0, 2))
    return jnp.where(deg > 0, lax.rsqrt(jnp.maximum(deg, 1.0)), 0.0)


def _l1_body(degp_ref, x_ref, w_ref, b_ref, outa_ref, outb_ref):
    ds = _ds_from_degp(degp_ref[...])
    sup = lax.dot_general(x_ref[...], w_ref[...],
                          (((1,), (1,)), ((), ())),
                          preferred_element_type=jnp.float32)
    res = (sup + b_ref[...][None, :]) * ds[:, None]
    outa_ref[...] = res[:, :DH]
    outb_ref[...] = res[:, DH:]


def _l2_body(degp_ref, p00_ref, p01_ref, p10_ref, p11_ref, w_ref, b_ref,
             outa_ref, outb_ref):
    ds = _ds_from_degp(degp_ref[...])
    agg = jnp.concatenate([p00_ref[...] + p10_ref[...],
                           p01_ref[...] + p11_ref[...]], axis=1)
    h = jnp.maximum(agg * ds[:, None], 0.0)
    sup = lax.dot_general(h, w_ref[...],
                          (((1,), (1,)), ((), ())),
                          preferred_element_type=jnp.float32)
    res = (sup + b_ref[...][None, :]) * ds[:, None]
    outa_ref[...] = res[:, :DH]
    outb_ref[...] = res[:, DH:]


def _fin_body(degp_ref, p00_ref, p01_ref, p10_ref, p11_ref, out_ref):
    ds = _ds_from_degp(degp_ref[...])
    agg = jnp.concatenate([p00_ref[...] + p10_ref[...],
                           p01_ref[...] + p11_ref[...]], axis=1)
    out_ref[...] = agg * ds[:, None]


_degp_spec = pl.BlockSpec((NC, _RB, 16), lambda i: (0, i, 0))
_row_spec = pl.BlockSpec((_RB, D), lambda i: (i, 0))
_half_spec = pl.BlockSpec((_RB, DH), lambda i: (i, 0))
_w_spec = pl.BlockSpec((D, D), lambda i: (0, 0))
_b_spec = pl.BlockSpec((D,), lambda i: (0,))
_out_struct = jax.ShapeDtypeStruct((N, D), jnp.float32)
_half_struct = jax.ShapeDtypeStruct((N, DH), jnp.float32)

_l1_call = pl.pallas_call(
    _l1_body, grid=(N // _RB,),
    in_specs=[_degp_spec, _row_spec, _w_spec, _b_spec],
    out_specs=[_half_spec, _half_spec],
    out_shape=[_half_struct, _half_struct])

_l2_call = pl.pallas_call(
    _l2_body, grid=(N // _RB,),
    in_specs=[_degp_spec, _half_spec, _half_spec, _half_spec, _half_spec,
              _w_spec, _b_spec],
    out_specs=[_half_spec, _half_spec],
    out_shape=[_half_struct, _half_struct])

_fin_call = pl.pallas_call(
    _fin_body, grid=(N // _RB,),
    in_specs=[_degp_spec, _half_spec, _half_spec, _half_spec, _half_spec],
    out_specs=_row_spec, out_shape=_out_struct)


def kernel(edge_index, feature_matrix, W1, b1, W2, b2):
    src = edge_index[0]
    dst = edge_index[1]
    src_p = jnp.concatenate(
        [src, jnp.zeros((EP - E,), jnp.int32)]).reshape(NC, NS, NCHUNK, CH)
    dst_p = jnp.concatenate(
        [dst, jnp.full((EP - E,), N, jnp.int32)]).reshape(NC, NS, NCHUNK, CH)

    degp = _deg_kernel(dst_p)[:, :N, :]

    ta, tb = _l1_call(degp, feature_matrix, W1, b1)
    p = _agg_kernel(ta, tb, src_p, dst_p)
    ta2, tb2 = _l2_call(degp, p[0, 0, :N], p[0, 1, :N],
                        p[1, 0, :N], p[1, 1, :N], W2, b2)
    q = _agg_kernel(ta2, tb2, src_p, dst_p)
    return _fin_call(degp, q[0, 0, :N], q[0, 1, :N],
                     q[1, 0, :N], q[1, 1, :N])


# speedup vs baseline: 2.2186x; 2.2186x over previous
import functools

import jax
import jax.numpy as jnp
from jax import lax
from jax.experimental import pallas as pl
from jax.experimental.pallas import tpu as pltpu
from jax.experimental.pallas import tpu_sc as plsc

N = 10000
E = 320000
D = 128
DH = D // 2

NC = 2
NS = 16
CH = 128
CHZ = 128
NCHUNK = 80
EP = NC * NS * NCHUNK * CH
R = 10240
RT = R // NS

_MESH = plsc.VectorSubcoreMesh(core_axis_name="c", subcore_axis_name="s")


@functools.partial(
    pl.kernel,
    out_type=jax.ShapeDtypeStruct((NC, R, 16), jnp.float32),
    mesh=_MESH,
    scratch_types=[
        pltpu.VMEM((NCHUNK, CH), jnp.int32),
        pltpu.VMEM((CH, 16), jnp.float32),
        pltpu.VMEM((CHZ, 16), jnp.float32),
        pltpu.VMEM_SHARED((R, 16), jnp.float32),
    ],
    compiler_params=pltpu.CompilerParams(use_tc_tiling_on_sc=False),
)
def _deg_kernel(dst_hbm, out_hbm, dst_v, ones_v, zb_v, acc_sh):
    c = lax.axis_index("c")
    s = lax.axis_index("s")
    e0 = jnp.where(lax.iota(jnp.int32, 16) == 0,
                   jnp.float32(1.0), jnp.float32(0.0))
    z16 = jnp.zeros((16,), jnp.float32)

    def init_row(j, _):
        ones_v[j, :] = e0
        return 0
    lax.fori_loop(0, CH, init_row, 0)

    def init_zrow(j, _):
        zb_v[j, :] = z16
        return 0
    lax.fori_loop(0, CHZ, init_zrow, 0)

    base = s * RT
    def zcp(k, _):
        pltpu.sync_copy(zb_v, acc_sh.at[pl.ds(base + k * CHZ, CHZ)])
        return 0
    lax.fori_loop(0, RT // CHZ, zcp, 0)
    plsc.subcore_barrier()

    pltpu.sync_copy(dst_hbm.at[c, s], dst_v)

    def scat(j, _):
        pltpu.sync_copy(ones_v, acc_sh.at[dst_v.at[j]], add=True)
        return 0
    lax.fori_loop(0, NCHUNK, scat, 0)
    plsc.subcore_barrier()

    pltpu.sync_copy(acc_sh.at[pl.ds(base, RT)], out_hbm.at[c, pl.ds(base, RT)])


@functools.partial(
    pl.kernel,
    out_type=jax.ShapeDtypeStruct((NC, 2, R, DH), jnp.float32),
    mesh=_MESH,
    scratch_types=[
        pltpu.VMEM((NCHUNK, CH), jnp.int32),
        pltpu.VMEM((NCHUNK, CH), jnp.int32),
        pltpu.VMEM((CH, DH), jnp.float32),
        pltpu.VMEM((CH, DH), jnp.float32),
        pltpu.VMEM((CHZ, DH), jnp.float32),
        pltpu.VMEM_SHARED((R, DH), jnp.float32),
        pltpu.VMEM_SHARED((R, DH), jnp.float32),
        pltpu.SemaphoreType.DMA,
        pltpu.SemaphoreType.DMA,
    ],
    compiler_params=pltpu.CompilerParams(use_tc_tiling_on_sc=False),
)
def _agg_kernel(ta_hbm, tb_hbm, src_hbm, dst_hbm, out_hbm,
                src_v, dst_v, rb0, rb1, zb_v, acc_sh, tbl_sh, sem0, sem1):
    c = lax.axis_index("c")
    s = lax.axis_index("s")
    z16 = jnp.zeros((16,), jnp.float32)

    def zrow(j, _):
        for k in range(DH // 16):
            zb_v[j, pl.ds(k * 16, 16)] = z16
        return 0
    lax.fori_loop(0, CHZ, zrow, 0)

    base = s * RT
    pltpu.sync_copy(src_hbm.at[c, s], src_v)
    pltpu.sync_copy(dst_hbm.at[c, s], dst_v)

    for phase, table in ((0, ta_hbm), (1, tb_hbm)):
        pltpu.sync_copy(table.at[pl.ds(base, RT)], tbl_sh.at[pl.ds(base, RT)])
        def zcp(k, _):
            pltpu.sync_copy(zb_v, acc_sh.at[pl.ds(base + k * CHZ, CHZ)])
            return 0
        lax.fori_loop(0, RT // CHZ, zcp, 0)
        plsc.subcore_barrier()

        pltpu.async_copy(tbl_sh.at[src_v.at[0]], rb0, sem0)
        pltpu.async_copy(tbl_sh.at[src_v.at[1]], rb1, sem1)

        def step(t, _):
            g = t * 2
            for b, (rb, sem) in enumerate(((rb0, sem0), (rb1, sem1))):
                j = g + b
                pltpu.make_async_copy(tbl_sh.at[src_v.at[j]], rb, sem).wait()
                pltpu.sync_copy(rb, acc_sh.at[dst_v.at[j]], add=True)

                @pl.when(j + 2 < NCHUNK)
                def _():
                    pltpu.async_copy(tbl_sh.at[src_v.at[j + 2]], rb, sem)
            return 0
        lax.fori_loop(0, NCHUNK // 2, step, 0)
        plsc.subcore_barrier()

        pltpu.sync_copy(acc_sh.at[pl.ds(base, RT)],
                        out_hbm.at[c, phase, pl.ds(base, RT)])


_RB = 1000


def _ds_from_degp(degp_blk):
    deg = jnp.sum(degp_blk, axis=(0, 2))
    return jnp.where(deg > 0, lax.rsqrt(jnp.maximum(deg, 1.0)), 0.0)


def _l1_body(degp_ref, x_ref, w_ref, b_ref, outa_ref, outb_ref):
    ds = _ds_from_degp(degp_ref[...])
    sup = lax.dot_general(x_ref[...], w_ref[...],
                          (((1,), (1,)), ((), ())),
                          preferred_element_type=jnp.float32)
    res = (sup + b_ref[...][None, :]) * ds[:, None]
    outa_ref[...] = res[:, :DH]
    outb_ref[...] = res[:, DH:]


def _l2_body(degp_ref, p00_ref, p01_ref, p10_ref, p11_ref, w_ref, b_ref,
             outa_ref, outb_ref):
    ds = _ds_from_degp(degp_ref[...])
    agg = jnp.concatenate([p00_ref[...] + p10_ref[...],
                           p01_ref[...] + p11_ref[...]], axis=1)
    h = jnp.maximum(agg * ds[:, None], 0.0)
    sup = lax.dot_general(h, w_ref[...],
                          (((1,), (1,)), ((), ())),
                          preferred_element_type=jnp.float32)
    res = (sup + b_ref[...][None, :]) * ds[:, None]
    outa_ref[...] = res[:, :DH]
    outb_ref[...] = res[:, DH:]


def _fin_body(degp_ref, p00_ref, p01_ref, p10_ref, p11_ref, out_ref):
    ds = _ds_from_degp(degp_ref[...])
    agg = jnp.concatenate([p00_ref[...] + p10_ref[...],
                           p01_ref[...] + p11_ref[...]], axis=1)
    out_ref[...] = agg * ds[:, None]


_degp_spec = pl.BlockSpec((NC, _RB, 16), lambda i: (0, i, 0))
_row_spec = pl.BlockSpec((_RB, D), lambda i: (i, 0))
_half_spec = pl.BlockSpec((_RB, DH), lambda i: (i, 0))
_w_spec = pl.BlockSpec((D, D), lambda i: (0, 0))
_b_spec = pl.BlockSpec((D,), lambda i: (0,))
_out_struct = jax.ShapeDtypeStruct((N, D), jnp.float32)
_half_struct = jax.ShapeDtypeStruct((R, DH), jnp.float32)

_l1_call = pl.pallas_call(
    _l1_body, grid=(N // _RB,),
    in_specs=[_degp_spec, _row_spec, _w_spec, _b_spec],
    out_specs=[_half_spec, _half_spec],
    out_shape=[_half_struct, _half_struct])

_l2_call = pl.pallas_call(
    _l2_body, grid=(N // _RB,),
    in_specs=[_degp_spec, _half_spec, _half_spec, _half_spec, _half_spec,
              _w_spec, _b_spec],
    out_specs=[_half_spec, _half_spec],
    out_shape=[_half_struct, _half_struct])

_fin_call = pl.pallas_call(
    _fin_body, grid=(N // _RB,),
    in_specs=[_degp_spec, _half_spec, _half_spec, _half_spec, _half_spec],
    out_specs=_row_spec, out_shape=_out_struct)


def kernel(edge_index, feature_matrix, W1, b1, W2, b2):
    src = edge_index[0]
    dst = edge_index[1]
    src_p = jnp.concatenate(
        [src, jnp.zeros((EP - E,), jnp.int32)]).reshape(NC, NS, NCHUNK, CH)
    dst_p = jnp.concatenate(
        [dst, jnp.full((EP - E,), N, jnp.int32)]).reshape(NC, NS, NCHUNK, CH)

    degp = _deg_kernel(dst_p)

    ta, tb = _l1_call(degp, feature_matrix, W1, b1)
    p = _agg_kernel(ta, tb, src_p, dst_p)
    ta2, tb2 = _l2_call(degp, p[0, 0], p[0, 1], p[1, 0], p[1, 1], W2, b2)
    q = _agg_kernel(ta2, tb2, src_p, dst_p)
    return _fin_call(degp, q[0, 0], q[0, 1], q[1, 0], q[1, 1])


# speedup vs baseline: 2.3904x; 1.0774x over previous
import functools

import jax
import jax.numpy as jnp
from jax import lax
from jax.experimental import pallas as pl
from jax.experimental.pallas import tpu as pltpu
from jax.experimental.pallas import tpu_sc as plsc

N = 10000
E = 320000
D = 128
DH = D // 2

NC = 2
NS = 16
CH = 96
CHZ = 32
NCHUNK = 108
EP = NC * NS * NCHUNK * CH
R = 10240
RT = R // NS
NTS = N // NS

_MESH = plsc.VectorSubcoreMesh(core_axis_name="c", subcore_axis_name="s")


@functools.partial(
    pl.kernel,
    out_type=jax.ShapeDtypeStruct((NC, R, 16), jnp.float32),
    mesh=_MESH,
    scratch_types=[
        pltpu.VMEM((NCHUNK, CH), jnp.int32),
        pltpu.VMEM((CH, 16), jnp.float32),
        pltpu.VMEM((CHZ, 16), jnp.float32),
        pltpu.VMEM_SHARED((R, 16), jnp.float32),
    ],
    compiler_params=pltpu.CompilerParams(use_tc_tiling_on_sc=False),
)
def _deg_kernel(dst_hbm, out_hbm, dst_v, ones_v, zb_v, acc_sh):
    c = lax.axis_index("c")
    s = lax.axis_index("s")
    e0 = jnp.where(lax.iota(jnp.int32, 16) == 0,
                   jnp.float32(1.0), jnp.float32(0.0))
    z16 = jnp.zeros((16,), jnp.float32)

    def init_row(j, _):
        ones_v[j, :] = e0
        return 0
    lax.fori_loop(0, CH, init_row, 0)

    def init_zrow(j, _):
        zb_v[j, :] = z16
        return 0
    lax.fori_loop(0, CHZ, init_zrow, 0)

    base = s * RT
    def zcp(k, _):
        pltpu.sync_copy(zb_v, acc_sh.at[pl.ds(base + k * CHZ, CHZ)])
        return 0
    lax.fori_loop(0, RT // CHZ, zcp, 0)
    plsc.subcore_barrier()

    pltpu.sync_copy(dst_hbm.at[c, s], dst_v)

    def scat(j, _):
        pltpu.sync_copy(ones_v, acc_sh.at[dst_v.at[j]], add=True)
        return 0
    lax.fori_loop(0, NCHUNK, scat, 0)
    plsc.subcore_barrier()

    pltpu.sync_copy(acc_sh.at[pl.ds(base, RT)], out_hbm.at[c, pl.ds(base, RT)])


@functools.partial(
    pl.kernel,
    out_type=jax.ShapeDtypeStruct((NC, 2, R, DH), jnp.float32),
    mesh=_MESH,
    scratch_types=[
        pltpu.VMEM((NCHUNK, CH), jnp.int32),
        pltpu.VMEM((NCHUNK, CH), jnp.int32),
        pltpu.VMEM((CH, DH), jnp.float32),
        pltpu.VMEM((CH, DH), jnp.float32),
        pltpu.VMEM((CH, DH), jnp.float32),
        pltpu.VMEM((CH, DH), jnp.float32),
        pltpu.VMEM((CHZ, DH), jnp.float32),
        pltpu.VMEM_SHARED((R, DH), jnp.float32),
        pltpu.VMEM_SHARED((N, DH), jnp.float32),
        pltpu.SemaphoreType.DMA,
        pltpu.SemaphoreType.DMA,
        pltpu.SemaphoreType.DMA,
        pltpu.SemaphoreType.DMA,
        pltpu.SemaphoreType.DMA,
        pltpu.SemaphoreType.DMA,
        pltpu.SemaphoreType.DMA,
        pltpu.SemaphoreType.DMA,
    ],
    compiler_params=pltpu.CompilerParams(use_tc_tiling_on_sc=False),
)
def _agg_kernel(ta_hbm, tb_hbm, src_hbm, dst_hbm, out_hbm,
                src_v, dst_v, rb0, rb1, rb2, rb3, zb_v, acc_sh, tbl_sh,
                g0, g1, g2, g3, s0, s1, s2, s3):
    c = lax.axis_index("c")
    s = lax.axis_index("s")
    z16 = jnp.zeros((16,), jnp.float32)

    def zrow(j, _):
        for k in range(DH // 16):
            zb_v[j, pl.ds(k * 16, 16)] = z16
        return 0
    lax.fori_loop(0, CHZ, zrow, 0)

    base = s * RT
    pltpu.sync_copy(src_hbm.at[c, s], src_v)
    pltpu.sync_copy(dst_hbm.at[c, s], dst_v)

    rbs = (rb0, rb1, rb2, rb3)
    gsem = (g0, g1, g2, g3)
    ssem = (s0, s1, s2, s3)

    for phase, table in ((0, ta_hbm), (1, tb_hbm)):
        pltpu.sync_copy(table.at[pl.ds(s * NTS, NTS)],
                        tbl_sh.at[pl.ds(s * NTS, NTS)])
        def zcp(k, _):
            pltpu.sync_copy(zb_v, acc_sh.at[pl.ds(base + k * CHZ, CHZ)])
            return 0
        lax.fori_loop(0, RT // CHZ, zcp, 0)
        plsc.subcore_barrier()

        pltpu.async_copy(tbl_sh.at[src_v.at[0]], rb0, g0)
        pltpu.async_copy(tbl_sh.at[src_v.at[1]], rb1, g1)

        def group(m, _):
            for b in range(4):
                j = 4 * m + b
                bg = (b + 2) % 4
                if b < 2:
                    @pl.when(m > 0)
                    def _():
                        pltpu.make_async_copy(
                            rbs[bg], acc_sh.at[dst_v.at[j - 2]],
                            ssem[bg]).wait()
                    pltpu.async_copy(tbl_sh.at[src_v.at[j + 2]],
                                     rbs[bg], gsem[bg])
                else:
                    pltpu.make_async_copy(
                        rbs[bg], acc_sh.at[dst_v.at[j - 2]], ssem[bg]).wait()

                    @pl.when(m < NCHUNK // 4 - 1)
                    def _():
                        pltpu.async_copy(tbl_sh.at[src_v.at[j + 2]],
                                         rbs[bg], gsem[bg])
                pltpu.make_async_copy(tbl_sh.at[src_v.at[j]],
                                      rbs[b], gsem[b]).wait()
                pltpu.async_copy(rbs[b], acc_sh.at[dst_v.at[j]], ssem[b],
                                 add=True)
            return 0
        lax.fori_loop(0, NCHUNK // 4, group, 0)
        pltpu.make_async_copy(rbs[2], acc_sh.at[dst_v.at[NCHUNK - 2]],
                              ssem[2]).wait()
        pltpu.make_async_copy(rbs[3], acc_sh.at[dst_v.at[NCHUNK - 1]],
                              ssem[3]).wait()
        plsc.subcore_barrier()

        pltpu.sync_copy(acc_sh.at[pl.ds(base, RT)],
                        out_hbm.at[c, phase, pl.ds(base, RT)])


_RB = 1000


def _ds_from_degp(degp_blk):
    deg = jnp.sum(degp_blk, axis=(0, 2))
    return jnp.where(deg > 0, lax.rsqrt(jnp.maximum(deg, 1.0)), 0.0)


def _l1_body(degp_ref, x_ref, w_ref, b_ref, outa_ref, outb_ref):
    ds = _ds_from_degp(degp_ref[...])
    sup = lax.dot_general(x_ref[...], w_ref[...],
                          (((1,), (1,)), ((), ())),
                          preferred_element_type=jnp.float32)
    res = (sup + b_ref[...][None, :]) * ds[:, None]
    outa_ref[...] = res[:, :DH]
    outb_ref[...] = res[:, DH:]


def _l2_body(degp_ref, p00_ref, p01_ref, p10_ref, p11_ref, w_ref, b_ref,
             outa_ref, outb_ref):
    ds = _ds_from_degp(degp_ref[...])
    agg = jnp.concatenate([p00_ref[...] + p10_ref[...],
                           p01_ref[...] + p11_ref[...]], axis=1)
    h = jnp.maximum(agg * ds[:, None], 0.0)
    sup = lax.dot_general(h, w_ref[...],
                          (((1,), (1,)), ((), ())),
                          preferred_element_type=jnp.float32)
    res = (sup + b_ref[...][None, :]) * ds[:, None]
    outa_ref[...] = res[:, :DH]
    outb_ref[...] = res[:, DH:]


def _fin_body(degp_ref, p00_ref, p01_ref, p10_ref, p11_ref, out_ref):
    ds = _ds_from_degp(degp_ref[...])
    agg = jnp.concatenate([p00_ref[...] + p10_ref[...],
                           p01_ref[...] + p11_ref[...]], axis=1)
    out_ref[...] = agg * ds[:, None]


_degp_spec = pl.BlockSpec((NC, _RB, 16), lambda i: (0, i, 0))
_row_spec = pl.BlockSpec((_RB, D), lambda i: (i, 0))
_half_spec = pl.BlockSpec((_RB, DH), lambda i: (i, 0))
_w_spec = pl.BlockSpec((D, D), lambda i: (0, 0))
_b_spec = pl.BlockSpec((D,), lambda i: (0,))
_out_struct = jax.ShapeDtypeStruct((N, D), jnp.float32)
_half_struct = jax.ShapeDtypeStruct((R, DH), jnp.float32)

_l1_call = pl.pallas_call(
    _l1_body, grid=(N // _RB,),
    in_specs=[_degp_spec, _row_spec, _w_spec, _b_spec],
    out_specs=[_half_spec, _half_spec],
    out_shape=[_half_struct, _half_struct])

_l2_call = pl.pallas_call(
    _l2_body, grid=(N // _RB,),
    in_specs=[_degp_spec, _half_spec, _half_spec, _half_spec, _half_spec,
              _w_spec, _b_spec],
    out_specs=[_half_spec, _half_spec],
    out_shape=[_half_struct, _half_struct])

_fin_call = pl.pallas_call(
    _fin_body, grid=(N // _RB,),
    in_specs=[_degp_spec, _half_spec, _half_spec, _half_spec, _half_spec],
    out_specs=_row_spec, out_shape=_out_struct)


def kernel(edge_index, feature_matrix, W1, b1, W2, b2):
    src = edge_index[0]
    dst = edge_index[1]
    src_p = jnp.concatenate(
        [src, jnp.zeros((EP - E,), jnp.int32)]).reshape(NC, NS, NCHUNK, CH)
    dst_p = jnp.concatenate(
        [dst, jnp.full((EP - E,), N, jnp.int32)]).reshape(NC, NS, NCHUNK, CH)

    degp = _deg_kernel(dst_p)

    ta, tb = _l1_call(degp, feature_matrix, W1, b1)
    p = _agg_kernel(ta, tb, src_p, dst_p)
    ta2, tb2 = _l2_call(degp, p[0, 0], p[0, 1], p[1, 0], p[1, 1], W2, b2)
    q = _agg_kernel(ta2, tb2, src_p, dst_p)
    return _fin_call(degp, q[0, 0], q[0, 1], q[1, 0], q[1, 1])
